# Initial kernel scaffold; baseline (speedup 1.0000x reference)
import jax
import jax.numpy as jnp
from jax.experimental import pallas as pl


def kernel(positions, species, senders, receivers, params):
    raise NotImplementedError("write your pallas kernel here")


# speedup vs baseline: 14.5983x; 14.5983x over previous
import functools

import jax
import jax.numpy as jnp
import numpy as np
from jax import lax
from jax.experimental import pallas as pl
from jax.experimental.pallas import tpu as pltpu
from jax.experimental.pallas import tpu_sc as plsc

N_NODES = 10000
N_EDGES = 160000
NUM_SPECIES = 5
EMB = 32
MUL = 64
NBASIS = 8
CUTOFF = 6.0
IRREPS_DIM = 16
LIDX = [0, 1, 1, 1, 2, 2, 2, 2, 2, 3, 3, 3, 3, 3, 3, 3]

T = 256
NT = 40
N_PAD = NT * T
EC = 512
SC_CHUNK = 1024
NW = 32
E_PAD = 163840
GEC = 2048

_F32 = jnp.float32
_I32 = jnp.int32


def _dot(a, b):
    return jnp.dot(a, b, preferred_element_type=_F32)


def _silu(x):
    return x * jax.nn.sigmoid(x)


def _sc_gather(table, idx):
    B = idx.shape[0]
    D = table.shape[1]
    bpw = B // NW
    nch = bpw // SC_CHUNK
    mesh = plsc.VectorSubcoreMesh(core_axis_name="c", subcore_axis_name="s")

    @functools.partial(
        pl.kernel,
        out_type=jax.ShapeDtypeStruct((B, D), _F32),
        mesh=mesh,
        compiler_params=pltpu.CompilerParams(use_tc_tiling_on_sc=False),
        scratch_types=[
            pltpu.VMEM((SC_CHUNK,), _I32),
            pltpu.VMEM((SC_CHUNK, D), _F32),
            pltpu.SemaphoreType.DMA,
        ],
    )
    def gk(table_hbm, idx_hbm, out_hbm, idx_v, rows_v, sem):
        wid = lax.axis_index("s") * 2 + lax.axis_index("c")
        for j in range(nch):
            base = wid * bpw + j * SC_CHUNK
            pltpu.sync_copy(idx_hbm.at[pl.ds(base, SC_CHUNK)], idx_v)
            pltpu.async_copy(table_hbm.at[idx_v], rows_v, sem).wait()
            pltpu.sync_copy(rows_v, out_hbm.at[pl.ds(base, SC_CHUNK)])

    return gk(table, idx)


def _geom_body(ps_ref, pr_ref, y_ref, b_ref):
    dx = pr_ref[:, 0:1] - ps_ref[:, 0:1]
    dy = pr_ref[:, 1:2] - ps_ref[:, 1:2]
    dz = pr_ref[:, 2:3] - ps_ref[:, 2:3]
    r = jnp.sqrt(dx * dx + dy * dy + dz * dz)
    inv = 1.0 / jnp.maximum(r, 1e-9)
    x = dx * inv
    y = dy * inv
    z = dz * inv
    s3 = 3.0 ** 0.5
    s15 = 15.0 ** 0.5
    s5 = 5.0 ** 0.5
    comps = [
        jnp.ones_like(x),
        s3 * x, s3 * y, s3 * z,
        s15 * x * y, s15 * y * z, 0.5 * s5 * (3.0 * z * z - 1.0),
        s15 * x * z, 0.5 * s15 * (x * x - y * y),
        (35.0 / 8.0) ** 0.5 * y * (3.0 * x * x - y * y),
        105.0 ** 0.5 * x * y * z,
        (21.0 / 8.0) ** 0.5 * y * (5.0 * z * z - 1.0),
        0.5 * 7.0 ** 0.5 * z * (5.0 * z * z - 3.0),
        (21.0 / 8.0) ** 0.5 * x * (5.0 * z * z - 1.0),
        0.5 * 105.0 ** 0.5 * z * (x * x - y * y),
        (35.0 / 8.0) ** 0.5 * x * (x * x - 3.0 * y * y),
    ]
    y_ref[...] = jnp.concatenate(comps, axis=1)
    rs = jnp.maximum(r, 1e-6)
    nmat = (lax.broadcasted_iota(_I32, (GEC, NBASIS), 1) + 1).astype(_F32)
    b = (2.0 / CUTOFF) ** 0.5 * jnp.sin(nmat * (np.pi / CUTOFF) * rs) / rs
    p = r / CUTOFF
    env = jnp.where(p < 1.0,
                    1.0 - 10.0 * p ** 3 + 15.0 * p ** 4 - 6.0 * p ** 5, 0.0)
    b_ref[...] = b * env


def _geom(ps, pr):
    return pl.pallas_call(
        _geom_body,
        grid=(E_PAD // GEC,),
        in_specs=[pl.BlockSpec((GEC, 16), lambda i: (i, 0))] * 2,
        out_specs=[pl.BlockSpec((GEC, 16), lambda i: (i, 0)),
                   pl.BlockSpec((GEC, NBASIS), lambda i: (i, 0))],
        out_shape=[jax.ShapeDtypeStruct((E_PAD, 16), _F32),
                   jax.ShapeDtypeStruct((E_PAD, NBASIS), _F32)],
    )(ps, pr)


def _emb_body(sp_ref, emb_ref, lin_ref, skip_ref, h_ref, sk_ref):
    sp = sp_ref[0, 0, :]
    oh = (sp[:, None] ==
          lax.broadcasted_iota(_I32, (T, 8), 1)).astype(_F32)
    scal = _dot(oh, emb_ref[...])
    h_ref[...] = _dot(scal, lin_ref[...])
    sk_ref[...] = _dot(scal, skip_ref[...])


def _emb(species3d, embp, lin0, skip0):
    return pl.pallas_call(
        _emb_body,
        grid=(NT,),
        in_specs=[pl.BlockSpec((1, 1, T), lambda i: (i, 0, 0)),
                  pl.BlockSpec((8, EMB), lambda i: (0, 0)),
                  pl.BlockSpec((EMB, MUL), lambda i: (0, 0)),
                  pl.BlockSpec((EMB, MUL), lambda i: (0, 0))],
        out_specs=[pl.BlockSpec((T, MUL), lambda i: (i, 0)),
                   pl.BlockSpec((T, MUL), lambda i: (i, 0))],
        out_shape=[jax.ShapeDtypeStruct((N_PAD, MUL), _F32),
                   jax.ShapeDtypeStruct((N_PAD, MUL), _F32)],
    )(species3d, embp, lin0, skip0)


def _c01_body(cs_ref, cc_ref, hs_hbm, bas_hbm, rcv_hbm, sk_ref,
              rw1_ref, rb1_ref, rw2c_ref, self0_ref, lin_ref, skip_ref,
              h_ref, sknext_ref, acc, hsb, basb, rcb, sem0, sem1, sem2):
    t = pl.program_id(0)
    acc[...] = jnp.zeros((T, MUL), _F32)
    c0 = cs_ref[t]
    cn = cc_ref[t]

    def chunk(k, carry):
        base = (c0 + k) * EC
        cp0 = pltpu.make_async_copy(hs_hbm.at[pl.ds(base, EC)], hsb, sem0)
        cp1 = pltpu.make_async_copy(bas_hbm.at[pl.ds(base, EC)], basb, sem1)
        cp2 = pltpu.make_async_copy(rcv_hbm.at[pl.ds(base, EC)], rcb, sem2)
        cp0.start(); cp1.start(); cp2.start()
        cp0.wait(); cp1.wait(); cp2.wait()
        z = _silu(_dot(basb[...], rw1_ref[...]) + rb1_ref[...])
        w0 = _dot(z, rw2c_ref[...])
        msg = hsb[...] * w0
        rloc = rcb[...] - t * T
        oh = (lax.broadcasted_iota(_I32, (T, EC), 0) ==
              rloc[None, :]).astype(_F32)
        acc[...] = acc[...] + _dot(oh, msg)
        return carry

    lax.fori_loop(0, cn, chunk, 0)
    out0 = _dot(acc[...], self0_ref[...]) + sk_ref[...]
    scal = _silu(out0)
    h_ref[...] = _dot(scal, lin_ref[...])
    sknext_ref[...] = _dot(scal, skip_ref[...])


def _c01(cs, cc, hs, bas, rcv, sk, rw1, rb1, rw2c, self0, lin_n, skip_n):
    grid_spec = pltpu.PrefetchScalarGridSpec(
        num_scalar_prefetch=2,
        grid=(NT,),
        in_specs=[
            pl.BlockSpec(memory_space=pl.ANY),
            pl.BlockSpec(memory_space=pl.ANY),
            pl.BlockSpec(memory_space=pl.ANY),
            pl.BlockSpec((T, MUL), lambda t, cs, cc: (t, 0)),
            pl.BlockSpec((NBASIS, 64), lambda t, cs, cc: (0, 0)),
            pl.BlockSpec((1, 64), lambda t, cs, cc: (0, 0)),
            pl.BlockSpec((64, MUL), lambda t, cs, cc: (0, 0)),
            pl.BlockSpec((MUL, MUL), lambda t, cs, cc: (0, 0)),
            pl.BlockSpec((MUL, MUL), lambda t, cs, cc: (0, 0)),
            pl.BlockSpec((MUL, MUL), lambda t, cs, cc: (0, 0)),
        ],
        out_specs=[pl.BlockSpec((T, MUL), lambda t, cs, cc: (t, 0)),
                   pl.BlockSpec((T, MUL), lambda t, cs, cc: (t, 0))],
        scratch_shapes=[
            pltpu.VMEM((T, MUL), _F32),
            pltpu.VMEM((EC, MUL), _F32),
            pltpu.VMEM((EC, NBASIS), _F32),
            pltpu.VMEM((EC,), _I32),
            pltpu.SemaphoreType.DMA,
            pltpu.SemaphoreType.DMA,
            pltpu.SemaphoreType.DMA,
        ],
    )
    return pl.pallas_call(
        _c01_body,
        grid_spec=grid_spec,
        out_shape=[jax.ShapeDtypeStruct((N_PAD, MUL), _F32),
                   jax.ShapeDtypeStruct((N_PAD, MUL), _F32)],
    )(cs, cc, hs, bas, rcv, sk, rw1, rb1, rw2c, self0, lin_n, skip_n)


def _c2_body(cs_ref, cc_ref, hs_hbm, bas_hbm, y_hbm, rcv_hbm, sk_ref,
             rw1_ref, rb1_ref, rw2p_ref, selfs_ref,
             out_ref, acc, hsb, basb, yb, rcb, sem0, sem1, sem2, sem3):
    t = pl.program_id(0)
    acc[...] = jnp.zeros((T, MUL * IRREPS_DIM), _F32)
    c0 = cs_ref[t]
    cn = cc_ref[t]

    def chunk(k, carry):
        base = (c0 + k) * EC
        cp0 = pltpu.make_async_copy(hs_hbm.at[pl.ds(base, EC)], hsb, sem0)
        cp1 = pltpu.make_async_copy(bas_hbm.at[pl.ds(base, EC)], basb, sem1)
        cp2 = pltpu.make_async_copy(y_hbm.at[pl.ds(base, EC)], yb, sem2)
        cp3 = pltpu.make_async_copy(rcv_hbm.at[pl.ds(base, EC)], rcb, sem3)
        cp0.start(); cp1.start(); cp2.start(); cp3.start()
        cp0.wait(); cp1.wait(); cp2.wait(); cp3.wait()
        z = _silu(_dot(basb[...], rw1_ref[...]) + rb1_ref[...])
        wp = _dot(z, rw2p_ref[...])
        hsv = hsb[...]
        yv = yb[...]
        hw = [hsv * wp[:, l * MUL:(l + 1) * MUL] for l in range(4)]
        msg = jnp.concatenate(
            [hw[LIDX[m]] * yv[:, m:m + 1] for m in range(IRREPS_DIM)], axis=1)
        rloc = rcb[...] - t * T
        oh = (lax.broadcasted_iota(_I32, (T, EC), 0) ==
              rloc[None, :]).astype(_F32)
        acc[...] = acc[...] + _dot(oh, msg)
        return carry

    lax.fori_loop(0, cn, chunk, 0)
    agg = acc[...]
    outs = [_dot(agg[:, m * MUL:(m + 1) * MUL],
                 selfs_ref[LIDX[m] * MUL:(LIDX[m] + 1) * MUL, :])
            for m in range(IRREPS_DIM)]
    out0 = outs[0] + sk_ref[...]
    scd = _silu(out0)
    gate = jax.nn.sigmoid(out0)
    out_ref[...] = jnp.concatenate(
        [scd] + [outs[m] * gate for m in range(1, IRREPS_DIM)], axis=1)


def _c2(cs, cc, hs, bas, yy, rcv, sk, rw1, rb1, rw2p, selfs):
    grid_spec = pltpu.PrefetchScalarGridSpec(
        num_scalar_prefetch=2,
        grid=(NT,),
        in_specs=[
            pl.BlockSpec(memory_space=pl.ANY),
            pl.BlockSpec(memory_space=pl.ANY),
            pl.BlockSpec(memory_space=pl.ANY),
            pl.BlockSpec(memory_space=pl.ANY),
            pl.BlockSpec((T, MUL), lambda t, cs, cc: (t, 0)),
            pl.BlockSpec((NBASIS, 64), lambda t, cs, cc: (0, 0)),
            pl.BlockSpec((1, 64), lambda t, cs, cc: (0, 0)),
            pl.BlockSpec((64, 256), lambda t, cs, cc: (0, 0)),
            pl.BlockSpec((256, MUL), lambda t, cs, cc: (0, 0)),
        ],
        out_specs=[pl.BlockSpec((T, MUL * IRREPS_DIM),
                                lambda t, cs, cc: (t, 0))],
        scratch_shapes=[
            pltpu.VMEM((T, MUL * IRREPS_DIM), _F32),
            pltpu.VMEM((EC, MUL), _F32),
            pltpu.VMEM((EC, NBASIS), _F32),
            pltpu.VMEM((EC, IRREPS_DIM), _F32),
            pltpu.VMEM((EC,), _I32),
            pltpu.SemaphoreType.DMA,
            pltpu.SemaphoreType.DMA,
            pltpu.SemaphoreType.DMA,
            pltpu.SemaphoreType.DMA,
        ],
    )
    return pl.pallas_call(
        _c2_body,
        grid_spec=grid_spec,
        out_shape=[jax.ShapeDtypeStruct((N_PAD, MUL * IRREPS_DIM), _F32)],
    )(cs, cc, hs, bas, yy, rcv, sk, rw1, rb1, rw2p, selfs)[0]


def kernel(positions, species, senders, receivers, params):
    receivers = receivers.astype(_I32)
    senders = senders.astype(_I32)
    perm = jnp.argsort(receivers)
    recv_s = receivers[perm]
    send_s = senders[perm]
    send_idx = jnp.zeros((E_PAD,), _I32).at[:N_EDGES].set(send_s)
    recv_idx = jnp.zeros((E_PAD,), _I32).at[:N_EDGES].set(recv_s)
    recv_m = jnp.full((E_PAD,), N_NODES, _I32).at[:N_EDGES].set(recv_s)
    bounds = jnp.arange(NT + 1, dtype=_I32) * T
    off = jnp.searchsorted(recv_s, bounds).astype(_I32)
    cs = off[:-1] // EC
    cc = (off[1:] + (EC - 1)) // EC - cs
    cc = jnp.maximum(cc, 0)

    lp0, lp1, lp2 = params['layers']
    embp = jnp.zeros((8, EMB), _F32).at[:NUM_SPECIES].set(params['embed'])
    species3d = jnp.zeros((N_PAD,), _I32).at[:N_NODES].set(
        species.astype(_I32)).reshape(NT, 1, T)
    pos16 = jnp.zeros((N_NODES, 16), _F32).at[:, :3].set(positions)

    ps = _sc_gather(pos16, send_idx)
    pr = _sc_gather(pos16, recv_idx)
    yy, bas = _geom(ps, pr)

    h0, sk0 = _emb(species3d, embp, lp0['lin'], lp0['skip'])

    def w01(lp):
        rw2c = lp['rw2'].reshape(64, MUL, 4)[:, :, 0]
        return (lp['rw1'], lp['rb1'].reshape(1, 64), rw2c, lp['self'][0])

    hs0 = _sc_gather(h0, send_idx)
    h1, sk1 = _c01(cs, cc, hs0, bas, recv_m, sk0, *w01(lp0),
                   lp1['lin'], lp1['skip'])
    hs1 = _sc_gather(h1, send_idx)
    h2, sk2 = _c01(cs, cc, hs1, bas, recv_m, sk1, *w01(lp1),
                   lp2['lin'], lp2['skip'])
    hs2 = _sc_gather(h2, send_idx)

    rw2p = lp2['rw2'].reshape(64, MUL, 4).transpose(0, 2, 1).reshape(64, 256)
    selfs = lp2['self'].reshape(4 * MUL, MUL)
    out2 = _c2(cs, cc, hs2, bas, yy, recv_m, sk2,
               lp2['rw1'], lp2['rb1'].reshape(1, 64), rw2p, selfs)
    return out2[:N_NODES].reshape(N_NODES, IRREPS_DIM, MUL).transpose(0, 2, 1)


# speedup vs baseline: 17.5063x; 1.1992x over previous
import functools

import jax
import jax.numpy as jnp
import numpy as np
from jax import lax
from jax.experimental import pallas as pl
from jax.experimental.pallas import tpu as pltpu
from jax.experimental.pallas import tpu_sc as plsc

N_NODES = 10000
N_EDGES = 160000
NUM_SPECIES = 5
EMB = 32
MUL = 64
NBASIS = 8
CUTOFF = 6.0
IRREPS_DIM = 16
LIDX = [0, 1, 1, 1, 2, 2, 2, 2, 2, 3, 3, 3, 3, 3, 3, 3]

T = 256
NT = 40
N_PAD = NT * T
T2 = 128
NT2 = N_PAD // T2
EC = 512
SC_CHUNK = 512
NW = 32
E_PAD = 163840
GEC = 2048

_F32 = jnp.float32
_BF16 = jnp.bfloat16
_I32 = jnp.int32


def _dot(a, b):
    return jnp.dot(a, b, preferred_element_type=_F32)


def _silu(x):
    return x * jax.nn.sigmoid(x)


def _sc_gather(table, idx):
    B = idx.shape[0]
    D = table.shape[1]
    bpw = B // NW
    nch = bpw // SC_CHUNK
    mesh = plsc.VectorSubcoreMesh(core_axis_name="c", subcore_axis_name="s")

    @functools.partial(
        pl.kernel,
        out_type=jax.ShapeDtypeStruct((B, D), _F32),
        mesh=mesh,
        compiler_params=pltpu.CompilerParams(use_tc_tiling_on_sc=False),
        scratch_types=[
            pltpu.VMEM((SC_CHUNK,), _I32),
            pltpu.VMEM((SC_CHUNK,), _I32),
            pltpu.VMEM((SC_CHUNK, D), _F32),
            pltpu.VMEM((SC_CHUNK, D), _F32),
            pltpu.SemaphoreType.DMA,
            pltpu.SemaphoreType.DMA,
            pltpu.SemaphoreType.DMA,
            pltpu.SemaphoreType.DMA,
            pltpu.SemaphoreType.DMA,
            pltpu.SemaphoreType.DMA,
        ],
    )
    def gk(table_hbm, idx_hbm, out_hbm, idx_a, idx_b, rows_a, rows_b,
           sia, sib, sga, sgb, swa, swb):
        wid = lax.axis_index("s") * 2 + lax.axis_index("c")
        idx_v = [idx_a, idx_b]
        rows_v = [rows_a, rows_b]
        sem_i = [sia, sib]
        sem_g = [sga, sgb]
        sem_w = [swa, swb]

        def base(j):
            return wid * bpw + j * SC_CHUNK

        def idx_copy(j, s):
            return pltpu.make_async_copy(
                idx_hbm.at[pl.ds(base(j), SC_CHUNK)], idx_v[s], sem_i[s])

        def out_copy(j, s):
            return pltpu.make_async_copy(
                rows_v[s], out_hbm.at[pl.ds(base(j), SC_CHUNK)], sem_w[s])

        idx_copy(0, 0).start()
        for j in range(nch):
            s = j % 2
            if j + 1 < nch:
                idx_copy(j + 1, 1 - s).start()
            idx_copy(j, s).wait()
            if j >= 2:
                out_copy(j - 2, s).wait()
            cg = pltpu.make_async_copy(
                table_hbm.at[idx_v[s]], rows_v[s], sem_g[s])
            cg.start()
            cg.wait()
            out_copy(j, s).start()
        for j in (nch - 2, nch - 1):
            out_copy(j, j % 2).wait()

    return gk(table, idx)


def _geom_body(ps_ref, pr_ref, y_ref, b_ref):
    dx = pr_ref[:, 0:1] - ps_ref[:, 0:1]
    dy = pr_ref[:, 1:2] - ps_ref[:, 1:2]
    dz = pr_ref[:, 2:3] - ps_ref[:, 2:3]
    r = jnp.sqrt(dx * dx + dy * dy + dz * dz)
    inv = 1.0 / jnp.maximum(r, 1e-9)
    x = dx * inv
    y = dy * inv
    z = dz * inv
    s3 = 3.0 ** 0.5
    s15 = 15.0 ** 0.5
    s5 = 5.0 ** 0.5
    comps = [
        jnp.ones_like(x),
        s3 * x, s3 * y, s3 * z,
        s15 * x * y, s15 * y * z, 0.5 * s5 * (3.0 * z * z - 1.0),
        s15 * x * z, 0.5 * s15 * (x * x - y * y),
        (35.0 / 8.0) ** 0.5 * y * (3.0 * x * x - y * y),
        105.0 ** 0.5 * x * y * z,
        (21.0 / 8.0) ** 0.5 * y * (5.0 * z * z - 1.0),
        0.5 * 7.0 ** 0.5 * z * (5.0 * z * z - 3.0),
        (21.0 / 8.0) ** 0.5 * x * (5.0 * z * z - 1.0),
        0.5 * 105.0 ** 0.5 * z * (x * x - y * y),
        (35.0 / 8.0) ** 0.5 * x * (x * x - 3.0 * y * y),
    ]
    y_ref[...] = jnp.concatenate(comps, axis=1)
    rs = jnp.maximum(r, 1e-6)
    nmat = (lax.broadcasted_iota(_I32, (GEC, NBASIS), 1) + 1).astype(_F32)
    b = (2.0 / CUTOFF) ** 0.5 * jnp.sin(nmat * (np.pi / CUTOFF) * rs) / rs
    p = r / CUTOFF
    env = jnp.where(p < 1.0,
                    1.0 - 10.0 * p ** 3 + 15.0 * p ** 4 - 6.0 * p ** 5, 0.0)
    b_ref[...] = b * env


def _geom(ps, pr):
    return pl.pallas_call(
        _geom_body,
        grid=(E_PAD // GEC,),
        in_specs=[pl.BlockSpec((GEC, 16), lambda i: (i, 0))] * 2,
        out_specs=[pl.BlockSpec((GEC, 16), lambda i: (i, 0)),
                   pl.BlockSpec((GEC, NBASIS), lambda i: (i, 0))],
        out_shape=[jax.ShapeDtypeStruct((E_PAD, 16), _F32),
                   jax.ShapeDtypeStruct((E_PAD, NBASIS), _F32)],
    )(ps, pr)


def _emb_body(sp_ref, emb_ref, lin_ref, skip_ref, h_ref, sk_ref):
    sp = sp_ref[0, 0, :]
    oh = (sp[:, None] ==
          lax.broadcasted_iota(_I32, (T, 8), 1)).astype(_F32)
    scal = _dot(oh, emb_ref[...])
    h_ref[...] = _dot(scal, lin_ref[...])
    sk_ref[...] = _dot(scal, skip_ref[...])


def _emb(species3d, embp, lin0, skip0):
    return pl.pallas_call(
        _emb_body,
        grid=(NT,),
        in_specs=[pl.BlockSpec((1, 1, T), lambda i: (i, 0, 0)),
                  pl.BlockSpec((8, EMB), lambda i: (0, 0)),
                  pl.BlockSpec((EMB, MUL), lambda i: (0, 0)),
                  pl.BlockSpec((EMB, MUL), lambda i: (0, 0))],
        out_specs=[pl.BlockSpec((T, MUL), lambda i: (i, 0)),
                   pl.BlockSpec((T, MUL), lambda i: (i, 0))],
        out_shape=[jax.ShapeDtypeStruct((N_PAD, MUL), _F32),
                   jax.ShapeDtypeStruct((N_PAD, MUL), _F32)],
    )(species3d, embp, lin0, skip0)


def _edge_loop(cn, c0, hbm_refs, bufs, sems, compute):
    def start(k, slot):
        base = (c0 + k) * EC
        for r, b, s in zip(hbm_refs, bufs, sems):
            pltpu.make_async_copy(
                r.at[pl.ds(base, EC)], b.at[slot], s.at[slot]).start()

    def wait(k, slot):
        base = (c0 + k) * EC
        for r, b, s in zip(hbm_refs, bufs, sems):
            pltpu.make_async_copy(
                r.at[pl.ds(base, EC)], b.at[slot], s.at[slot]).wait()

    @pl.when(cn > 0)
    def _():
        start(0, 0)

    def body(k, carry):
        slot = lax.rem(k, 2)

        @pl.when(k + 1 < cn)
        def _():
            start(k + 1, 1 - slot)

        wait(k, slot)
        compute(slot)
        return carry

    lax.fori_loop(0, cn, body, 0)


def _c01_body(cs_ref, cc_ref, hs_hbm, bas_hbm, rcv_hbm, sk_ref,
              rw1_ref, rb1_ref, rw2c_ref, self0_ref, lin_ref, skip_ref,
              h_ref, sknext_ref, acc, hsb, basb, rcb, sh, sb, sr):
    t = pl.program_id(0)
    acc[...] = jnp.zeros((T, MUL), _F32)
    c0 = cs_ref[t]
    cn = cc_ref[t]

    def compute(slot):
        z = _silu(_dot(basb[slot], rw1_ref[...]) + rb1_ref[...])
        w0 = _dot(z, rw2c_ref[...])
        msg = (hsb[slot] * w0).astype(_BF16)
        rloc = rcb[slot] - t * T
        oh = (lax.broadcasted_iota(_I32, (T, EC), 0) ==
              rloc[None, :]).astype(_BF16)
        acc[...] = acc[...] + _dot(oh, msg)

    _edge_loop(cn, c0, [hs_hbm, bas_hbm, rcv_hbm], [hsb, basb, rcb],
               [sh, sb, sr], compute)

    out0 = _dot(acc[...], self0_ref[...]) + sk_ref[...]
    scal = _silu(out0)
    h_ref[...] = _dot(scal, lin_ref[...])
    sknext_ref[...] = _dot(scal, skip_ref[...])


def _c01(cs, cc, hs, bas, rcv, sk, rw1, rb1, rw2c, self0, lin_n, skip_n):
    grid_spec = pltpu.PrefetchScalarGridSpec(
        num_scalar_prefetch=2,
        grid=(NT,),
        in_specs=[
            pl.BlockSpec(memory_space=pl.ANY),
            pl.BlockSpec(memory_space=pl.ANY),
            pl.BlockSpec(memory_space=pl.ANY),
            pl.BlockSpec((T, MUL), lambda t, cs, cc: (t, 0)),
            pl.BlockSpec((NBASIS, 64), lambda t, cs, cc: (0, 0)),
            pl.BlockSpec((1, 64), lambda t, cs, cc: (0, 0)),
            pl.BlockSpec((64, MUL), lambda t, cs, cc: (0, 0)),
            pl.BlockSpec((MUL, MUL), lambda t, cs, cc: (0, 0)),
            pl.BlockSpec((MUL, MUL), lambda t, cs, cc: (0, 0)),
            pl.BlockSpec((MUL, MUL), lambda t, cs, cc: (0, 0)),
        ],
        out_specs=[pl.BlockSpec((T, MUL), lambda t, cs, cc: (t, 0)),
                   pl.BlockSpec((T, MUL), lambda t, cs, cc: (t, 0))],
        scratch_shapes=[
            pltpu.VMEM((T, MUL), _F32),
            pltpu.VMEM((2, EC, MUL), _F32),
            pltpu.VMEM((2, EC, NBASIS), _F32),
            pltpu.VMEM((2, EC), _I32),
            pltpu.SemaphoreType.DMA((2,)),
            pltpu.SemaphoreType.DMA((2,)),
            pltpu.SemaphoreType.DMA((2,)),
        ],
    )
    return pl.pallas_call(
        _c01_body,
        grid_spec=grid_spec,
        out_shape=[jax.ShapeDtypeStruct((N_PAD, MUL), _F32),
                   jax.ShapeDtypeStruct((N_PAD, MUL), _F32)],
    )(cs, cc, hs, bas, rcv, sk, rw1, rb1, rw2c, self0, lin_n, skip_n)


def _c2_body(cs_ref, cc_ref, hs_hbm, bas_hbm, y_hbm, rcv_hbm, sk_ref,
             rw1_ref, rb1_ref, rw2p_ref, selfs_ref,
             out_ref, acc, hsb, basb, yb, rcb, sh, sb, sy, sr):
    t = pl.program_id(0)
    acc[...] = jnp.zeros((T2, MUL * IRREPS_DIM), _F32)
    c0 = cs_ref[t]
    cn = cc_ref[t]

    def compute(slot):
        z = _silu(_dot(basb[slot], rw1_ref[...]) + rb1_ref[...])
        wp = _dot(z, rw2p_ref[...])
        hsv = hsb[slot]
        yv = yb[slot]
        hw = [hsv * wp[:, l * MUL:(l + 1) * MUL] for l in range(4)]
        msg = jnp.concatenate(
            [hw[LIDX[m]] * yv[:, m:m + 1] for m in range(IRREPS_DIM)],
            axis=1).astype(_BF16)
        rloc = rcb[slot] - t * T2
        oh = (lax.broadcasted_iota(_I32, (T2, EC), 0) ==
              rloc[None, :]).astype(_BF16)
        acc[...] = acc[...] + _dot(oh, msg)

    _edge_loop(cn, c0, [hs_hbm, bas_hbm, y_hbm, rcv_hbm],
               [hsb, basb, yb, rcb], [sh, sb, sy, sr], compute)

    agg = acc[...]
    outs = [_dot(agg[:, m * MUL:(m + 1) * MUL],
                 selfs_ref[LIDX[m] * MUL:(LIDX[m] + 1) * MUL, :])
            for m in range(IRREPS_DIM)]
    out0 = outs[0] + sk_ref[...]
    scd = _silu(out0)
    gate = jax.nn.sigmoid(out0)
    out_ref[...] = jnp.concatenate(
        [scd] + [outs[m] * gate for m in range(1, IRREPS_DIM)], axis=1)


def _c2(cs, cc, hs, bas, yy, rcv, sk, rw1, rb1, rw2p, selfs):
    grid_spec = pltpu.PrefetchScalarGridSpec(
        num_scalar_prefetch=2,
        grid=(NT2,),
        in_specs=[
            pl.BlockSpec(memory_space=pl.ANY),
            pl.BlockSpec(memory_space=pl.ANY),
            pl.BlockSpec(memory_space=pl.ANY),
            pl.BlockSpec(memory_space=pl.ANY),
            pl.BlockSpec((T2, MUL), lambda t, cs, cc: (t, 0)),
            pl.BlockSpec((NBASIS, 64), lambda t, cs, cc: (0, 0)),
            pl.BlockSpec((1, 64), lambda t, cs, cc: (0, 0)),
            pl.BlockSpec((64, 256), lambda t, cs, cc: (0, 0)),
            pl.BlockSpec((256, MUL), lambda t, cs, cc: (0, 0)),
        ],
        out_specs=[pl.BlockSpec((T2, MUL * IRREPS_DIM),
                                lambda t, cs, cc: (t, 0))],
        scratch_shapes=[
            pltpu.VMEM((T2, MUL * IRREPS_DIM), _F32),
            pltpu.VMEM((2, EC, MUL), _F32),
            pltpu.VMEM((2, EC, NBASIS), _F32),
            pltpu.VMEM((2, EC, IRREPS_DIM), _F32),
            pltpu.VMEM((2, EC), _I32),
            pltpu.SemaphoreType.DMA((2,)),
            pltpu.SemaphoreType.DMA((2,)),
            pltpu.SemaphoreType.DMA((2,)),
            pltpu.SemaphoreType.DMA((2,)),
        ],
    )
    return pl.pallas_call(
        _c2_body,
        grid_spec=grid_spec,
        out_shape=[jax.ShapeDtypeStruct((N_PAD, MUL * IRREPS_DIM), _F32)],
    )(cs, cc, hs, bas, yy, rcv, sk, rw1, rb1, rw2p, selfs)[0]


def kernel(positions, species, senders, receivers, params):
    receivers = receivers.astype(_I32)
    senders = senders.astype(_I32)
    perm = jnp.argsort(receivers)
    recv_s = receivers[perm]
    send_s = senders[perm]
    send_idx = jnp.zeros((E_PAD,), _I32).at[:N_EDGES].set(send_s)
    recv_idx = jnp.zeros((E_PAD,), _I32).at[:N_EDGES].set(recv_s)
    recv_m = jnp.full((E_PAD,), N_NODES, _I32).at[:N_EDGES].set(recv_s)

    def chunk_ranges(tile):
        nt = N_PAD // tile
        bounds = jnp.arange(nt + 1, dtype=_I32) * tile
        off = jnp.searchsorted(recv_s, bounds).astype(_I32)
        cs = off[:-1] // EC
        cc = jnp.maximum((off[1:] + (EC - 1)) // EC - cs, 0)
        return cs, cc

    cs1, cc1 = chunk_ranges(T)
    cs2, cc2 = chunk_ranges(T2)

    lp0, lp1, lp2 = params['layers']
    embp = jnp.zeros((8, EMB), _F32).at[:NUM_SPECIES].set(params['embed'])
    species3d = jnp.zeros((N_PAD,), _I32).at[:N_NODES].set(
        species.astype(_I32)).reshape(NT, 1, T)
    pos16 = jnp.zeros((N_NODES, 16), _F32).at[:, :3].set(positions)

    ps = _sc_gather(pos16, send_idx)
    pr = _sc_gather(pos16, recv_idx)
    yy, bas = _geom(ps, pr)

    h0, sk0 = _emb(species3d, embp, lp0['lin'], lp0['skip'])

    def w01(lp):
        rw2c = lp['rw2'].reshape(64, MUL, 4)[:, :, 0]
        return (lp['rw1'], lp['rb1'].reshape(1, 64), rw2c, lp['self'][0])

    hs0 = _sc_gather(h0, send_idx)
    h1, sk1 = _c01(cs1, cc1, hs0, bas, recv_m, sk0, *w01(lp0),
                   lp1['lin'], lp1['skip'])
    hs1 = _sc_gather(h1, send_idx)
    h2, sk2 = _c01(cs1, cc1, hs1, bas, recv_m, sk1, *w01(lp1),
                   lp2['lin'], lp2['skip'])
    hs2 = _sc_gather(h2, send_idx)

    rw2p = lp2['rw2'].reshape(64, MUL, 4).transpose(0, 2, 1).reshape(64, 256)
    selfs = lp2['self'].reshape(4 * MUL, MUL)
    out2 = _c2(cs2, cc2, hs2, bas, yy, recv_m, sk2,
               lp2['rw1'], lp2['rb1'].reshape(1, 64), rw2p, selfs)
    return out2[:N_NODES].reshape(N_NODES, IRREPS_DIM, MUL).transpose(0, 2, 1)


# speedup vs baseline: 22.6889x; 1.2960x over previous
import functools

import jax
import jax.numpy as jnp
import numpy as np
from jax import lax
from jax.experimental import pallas as pl
from jax.experimental.pallas import tpu as pltpu
from jax.experimental.pallas import tpu_sc as plsc

N_NODES = 10000
N_EDGES = 160000
NUM_SPECIES = 5
EMB = 32
MUL = 64
NBASIS = 8
CUTOFF = 6.0
IRREPS_DIM = 16
LIDX = [0, 1, 1, 1, 2, 2, 2, 2, 2, 3, 3, 3, 3, 3, 3, 3]

T = 256
NT = 40
N_PAD = NT * T
T2 = 128
NT2 = N_PAD // T2
EC = 512
SC_CHUNK = 512
NW = 32
E_PAD = 163840
GEC = 2048

_F32 = jnp.float32
_BF16 = jnp.bfloat16
_I32 = jnp.int32


def _dot(a, b):
    return jnp.dot(a, b, preferred_element_type=_F32)


def _silu(x):
    return x * jax.nn.sigmoid(x)


def _sc_gather(table, idx):
    B = idx.shape[0]
    D = table.shape[1]
    bpw = B // NW
    nch = bpw // SC_CHUNK
    mesh = plsc.VectorSubcoreMesh(core_axis_name="c", subcore_axis_name="s")

    @functools.partial(
        pl.kernel,
        out_type=jax.ShapeDtypeStruct((B, D), _F32),
        mesh=mesh,
        compiler_params=pltpu.CompilerParams(use_tc_tiling_on_sc=False),
        scratch_types=[
            pltpu.VMEM((SC_CHUNK,), _I32),
            pltpu.VMEM((SC_CHUNK,), _I32),
            pltpu.VMEM((SC_CHUNK, D), _F32),
            pltpu.VMEM((SC_CHUNK, D), _F32),
            pltpu.SemaphoreType.DMA,
            pltpu.SemaphoreType.DMA,
            pltpu.SemaphoreType.DMA,
            pltpu.SemaphoreType.DMA,
            pltpu.SemaphoreType.DMA,
            pltpu.SemaphoreType.DMA,
        ],
    )
    def gk(table_hbm, idx_hbm, out_hbm, idx_a, idx_b, rows_a, rows_b,
           sia, sib, sga, sgb, swa, swb):
        wid = lax.axis_index("s") * 2 + lax.axis_index("c")
        idx_v = [idx_a, idx_b]
        rows_v = [rows_a, rows_b]
        sem_i = [sia, sib]
        sem_g = [sga, sgb]
        sem_w = [swa, swb]

        def base(j):
            return wid * bpw + j * SC_CHUNK

        def idx_copy(j, s):
            return pltpu.make_async_copy(
                idx_hbm.at[pl.ds(base(j), SC_CHUNK)], idx_v[s], sem_i[s])

        def out_copy(j, s):
            return pltpu.make_async_copy(
                rows_v[s], out_hbm.at[pl.ds(base(j), SC_CHUNK)], sem_w[s])

        idx_copy(0, 0).start()
        for j in range(nch):
            s = j % 2
            if j + 1 < nch:
                idx_copy(j + 1, 1 - s).start()
            idx_copy(j, s).wait()
            if j >= 2:
                out_copy(j - 2, s).wait()
            cg = pltpu.make_async_copy(
                table_hbm.at[idx_v[s]], rows_v[s], sem_g[s])
            cg.start()
            cg.wait()
            out_copy(j, s).start()
        for j in (nch - 2, nch - 1):
            out_copy(j, j % 2).wait()

    return gk(table, idx)


def _geom_body(ps_ref, pr_ref, y_ref, b_ref):
    dx = pr_ref[0:1, :] - ps_ref[0:1, :]
    dy = pr_ref[1:2, :] - ps_ref[1:2, :]
    dz = pr_ref[2:3, :] - ps_ref[2:3, :]
    r = jnp.sqrt(dx * dx + dy * dy + dz * dz)
    inv = 1.0 / jnp.maximum(r, 1e-9)
    x = dx * inv
    y = dy * inv
    z = dz * inv
    s3 = 3.0 ** 0.5
    s15 = 15.0 ** 0.5
    s5 = 5.0 ** 0.5
    comps = [
        jnp.ones_like(x),
        s3 * x, s3 * y, s3 * z,
        s15 * x * y, s15 * y * z, 0.5 * s5 * (3.0 * z * z - 1.0),
        s15 * x * z, 0.5 * s15 * (x * x - y * y),
        (35.0 / 8.0) ** 0.5 * y * (3.0 * x * x - y * y),
        105.0 ** 0.5 * x * y * z,
        (21.0 / 8.0) ** 0.5 * y * (5.0 * z * z - 1.0),
        0.5 * 7.0 ** 0.5 * z * (5.0 * z * z - 3.0),
        (21.0 / 8.0) ** 0.5 * x * (5.0 * z * z - 1.0),
        0.5 * 105.0 ** 0.5 * z * (x * x - y * y),
        (35.0 / 8.0) ** 0.5 * x * (x * x - 3.0 * y * y),
    ]
    y_ref[...] = jnp.concatenate(comps, axis=0)
    rs = jnp.maximum(r, 1e-6)
    p = r / CUTOFF
    env = jnp.where(p < 1.0,
                    1.0 - 10.0 * p ** 3 + 15.0 * p ** 4 - 6.0 * p ** 5, 0.0)
    scale = (2.0 / CUTOFF) ** 0.5
    arg = (np.pi / CUTOFF) * rs
    envr = scale * env / rs
    b_ref[...] = jnp.concatenate(
        [jnp.sin(float(n) * arg) * envr for n in range(1, NBASIS + 1)],
        axis=0)


def _geom(psT, prT):
    yT, bT = pl.pallas_call(
        _geom_body,
        grid=(E_PAD // GEC,),
        in_specs=[pl.BlockSpec((16, GEC), lambda i: (0, i))] * 2,
        out_specs=[pl.BlockSpec((16, GEC), lambda i: (0, i)),
                   pl.BlockSpec((NBASIS, GEC), lambda i: (0, i))],
        out_shape=[jax.ShapeDtypeStruct((16, E_PAD), _F32),
                   jax.ShapeDtypeStruct((NBASIS, E_PAD), _F32)],
    )(psT, prT)
    return yT, bT


def _emb_body(sp_ref, emb_ref, lin_ref, skip_ref, h_ref, sk_ref):
    sp = sp_ref[0, 0, :]
    oh = (sp[:, None] ==
          lax.broadcasted_iota(_I32, (T, 8), 1)).astype(_F32)
    scal = _dot(oh, emb_ref[...])
    h_ref[...] = _dot(scal, lin_ref[...])
    sk_ref[...] = _dot(scal, skip_ref[...])


def _emb(species3d, embp, lin0, skip0):
    return pl.pallas_call(
        _emb_body,
        grid=(NT,),
        in_specs=[pl.BlockSpec((1, 1, T), lambda i: (i, 0, 0)),
                  pl.BlockSpec((8, EMB), lambda i: (0, 0)),
                  pl.BlockSpec((EMB, MUL), lambda i: (0, 0)),
                  pl.BlockSpec((EMB, MUL), lambda i: (0, 0))],
        out_specs=[pl.BlockSpec((T, MUL), lambda i: (i, 0)),
                   pl.BlockSpec((T, MUL), lambda i: (i, 0))],
        out_shape=[jax.ShapeDtypeStruct((N_PAD, MUL), _F32),
                   jax.ShapeDtypeStruct((N_PAD, MUL), _F32)],
    )(species3d, embp, lin0, skip0)


def _edge_loop(cn, c0, hbm_refs, bufs, sems, compute):
    def start(k, slot):
        base = (c0 + k) * EC
        for r, b, s in zip(hbm_refs, bufs, sems):
            pltpu.make_async_copy(
                r.at[pl.ds(base, EC)], b.at[slot], s.at[slot]).start()

    def wait(k, slot):
        base = (c0 + k) * EC
        for r, b, s in zip(hbm_refs, bufs, sems):
            pltpu.make_async_copy(
                r.at[pl.ds(base, EC)], b.at[slot], s.at[slot]).wait()

    @pl.when(cn > 0)
    def _():
        start(0, 0)

    def body(k, carry):
        slot = lax.rem(k, 2)

        @pl.when(k + 1 < cn)
        def _():
            start(k + 1, 1 - slot)

        wait(k, slot)
        compute(slot)
        return carry

    lax.fori_loop(0, cn, body, 0)


def _c01_body(cs_ref, cc_ref, hs_hbm, bas_hbm, rcv_hbm, sk_ref,
              rw1_ref, rb1_ref, rw2c_ref, self0_ref, lin_ref, skip_ref,
              h_ref, sknext_ref, acc, hsb, basb, rcb, sh, sb, sr):
    t = pl.program_id(0)
    acc[...] = jnp.zeros((T, MUL), _F32)
    c0 = cs_ref[t]
    cn = cc_ref[t]

    def compute(slot):
        z = _silu(_dot(basb[slot], rw1_ref[...]) + rb1_ref[...])
        w0 = _dot(z, rw2c_ref[...])
        msg = (hsb[slot] * w0).astype(_BF16)
        rloc = rcb[slot] - t * T
        oh = (lax.broadcasted_iota(_I32, (T, EC), 0) ==
              rloc[None, :]).astype(_BF16)
        acc[...] = acc[...] + _dot(oh, msg)

    _edge_loop(cn, c0, [hs_hbm, bas_hbm, rcv_hbm], [hsb, basb, rcb],
               [sh, sb, sr], compute)

    out0 = _dot(acc[...], self0_ref[...]) + sk_ref[...]
    scal = _silu(out0)
    h_ref[...] = _dot(scal, lin_ref[...])
    sknext_ref[...] = _dot(scal, skip_ref[...])


def _c01(cs, cc, hs, bas, rcv, sk, rw1, rb1, rw2c, self0, lin_n, skip_n):
    grid_spec = pltpu.PrefetchScalarGridSpec(
        num_scalar_prefetch=2,
        grid=(NT,),
        in_specs=[
            pl.BlockSpec(memory_space=pl.ANY),
            pl.BlockSpec(memory_space=pl.ANY),
            pl.BlockSpec(memory_space=pl.ANY),
            pl.BlockSpec((T, MUL), lambda t, cs, cc: (t, 0)),
            pl.BlockSpec((NBASIS, 64), lambda t, cs, cc: (0, 0)),
            pl.BlockSpec((1, 64), lambda t, cs, cc: (0, 0)),
            pl.BlockSpec((64, MUL), lambda t, cs, cc: (0, 0)),
            pl.BlockSpec((MUL, MUL), lambda t, cs, cc: (0, 0)),
            pl.BlockSpec((MUL, MUL), lambda t, cs, cc: (0, 0)),
            pl.BlockSpec((MUL, MUL), lambda t, cs, cc: (0, 0)),
        ],
        out_specs=[pl.BlockSpec((T, MUL), lambda t, cs, cc: (t, 0)),
                   pl.BlockSpec((T, MUL), lambda t, cs, cc: (t, 0))],
        scratch_shapes=[
            pltpu.VMEM((T, MUL), _F32),
            pltpu.VMEM((2, EC, MUL), _F32),
            pltpu.VMEM((2, EC, NBASIS), _F32),
            pltpu.VMEM((2, EC), _I32),
            pltpu.SemaphoreType.DMA((2,)),
            pltpu.SemaphoreType.DMA((2,)),
            pltpu.SemaphoreType.DMA((2,)),
        ],
    )
    return pl.pallas_call(
        _c01_body,
        grid_spec=grid_spec,
        out_shape=[jax.ShapeDtypeStruct((N_PAD, MUL), _F32),
                   jax.ShapeDtypeStruct((N_PAD, MUL), _F32)],
    )(cs, cc, hs, bas, rcv, sk, rw1, rb1, rw2c, self0, lin_n, skip_n)


def _c2_body(cs_ref, cc_ref, hs_hbm, bas_hbm, y_hbm, rcv_hbm, sk_ref,
             rw1_ref, rb1_ref, rw2p_ref, selfs_ref,
             out_ref, acc, hsb, basb, yb, rcb, sh, sb, sy, sr):
    t = pl.program_id(0)
    acc[...] = jnp.zeros((T2, MUL * IRREPS_DIM), _F32)
    c0 = cs_ref[t]
    cn = cc_ref[t]

    def compute(slot):
        z = _silu(_dot(basb[slot], rw1_ref[...]) + rb1_ref[...])
        wp = _dot(z, rw2p_ref[...])
        hsv = hsb[slot]
        yv = yb[slot]
        hw = [hsv * wp[:, l * MUL:(l + 1) * MUL] for l in range(4)]
        msg = jnp.concatenate(
            [hw[LIDX[m]] * yv[:, m:m + 1] for m in range(IRREPS_DIM)],
            axis=1).astype(_BF16)
        rloc = rcb[slot] - t * T2
        oh = (lax.broadcasted_iota(_I32, (T2, EC), 0) ==
              rloc[None, :]).astype(_BF16)
        acc[...] = acc[...] + _dot(oh, msg)

    _edge_loop(cn, c0, [hs_hbm, bas_hbm, y_hbm, rcv_hbm],
               [hsb, basb, yb, rcb], [sh, sb, sy, sr], compute)

    agg = acc[...]
    outs = [_dot(agg[:, m * MUL:(m + 1) * MUL],
                 selfs_ref[LIDX[m] * MUL:(LIDX[m] + 1) * MUL, :])
            for m in range(IRREPS_DIM)]
    out0 = outs[0] + sk_ref[...]
    scd = _silu(out0)
    gate = jax.nn.sigmoid(out0)
    out_ref[...] = jnp.concatenate(
        [scd] + [outs[m] * gate for m in range(1, IRREPS_DIM)], axis=1)


def _c2(cs, cc, hs, bas, yy, rcv, sk, rw1, rb1, rw2p, selfs):
    grid_spec = pltpu.PrefetchScalarGridSpec(
        num_scalar_prefetch=2,
        grid=(NT2,),
        in_specs=[
            pl.BlockSpec(memory_space=pl.ANY),
            pl.BlockSpec(memory_space=pl.ANY),
            pl.BlockSpec(memory_space=pl.ANY),
            pl.BlockSpec(memory_space=pl.ANY),
            pl.BlockSpec((T2, MUL), lambda t, cs, cc: (t, 0)),
            pl.BlockSpec((NBASIS, 64), lambda t, cs, cc: (0, 0)),
            pl.BlockSpec((1, 64), lambda t, cs, cc: (0, 0)),
            pl.BlockSpec((64, 256), lambda t, cs, cc: (0, 0)),
            pl.BlockSpec((256, MUL), lambda t, cs, cc: (0, 0)),
        ],
        out_specs=[pl.BlockSpec((T2, MUL * IRREPS_DIM),
                                lambda t, cs, cc: (t, 0))],
        scratch_shapes=[
            pltpu.VMEM((T2, MUL * IRREPS_DIM), _F32),
            pltpu.VMEM((2, EC, MUL), _F32),
            pltpu.VMEM((2, EC, NBASIS), _F32),
            pltpu.VMEM((2, EC, IRREPS_DIM), _F32),
            pltpu.VMEM((2, EC), _I32),
            pltpu.SemaphoreType.DMA((2,)),
            pltpu.SemaphoreType.DMA((2,)),
            pltpu.SemaphoreType.DMA((2,)),
            pltpu.SemaphoreType.DMA((2,)),
        ],
    )
    return pl.pallas_call(
        _c2_body,
        grid_spec=grid_spec,
        out_shape=[jax.ShapeDtypeStruct((N_PAD, MUL * IRREPS_DIM), _F32)],
    )(cs, cc, hs, bas, yy, rcv, sk, rw1, rb1, rw2p, selfs)[0]


def kernel(positions, species, senders, receivers, params):
    receivers = receivers.astype(_I32)
    senders = senders.astype(_I32)
    perm = jnp.argsort(receivers)
    recv_s = receivers[perm]
    send_s = senders[perm]
    send_idx = jnp.zeros((E_PAD,), _I32).at[:N_EDGES].set(send_s)
    recv_idx = jnp.zeros((E_PAD,), _I32).at[:N_EDGES].set(recv_s)
    recv_m = jnp.full((E_PAD,), N_NODES, _I32).at[:N_EDGES].set(recv_s)

    def chunk_ranges(tile):
        nt = N_PAD // tile
        bounds = jnp.arange(nt + 1, dtype=_I32) * tile
        off = jnp.searchsorted(recv_s, bounds).astype(_I32)
        cs = off[:-1] // EC
        cc = jnp.maximum((off[1:] + (EC - 1)) // EC - cs, 0)
        return cs, cc

    cs1, cc1 = chunk_ranges(T)
    cs2, cc2 = chunk_ranges(T2)

    lp0, lp1, lp2 = params['layers']
    embp = jnp.zeros((8, EMB), _F32).at[:NUM_SPECIES].set(params['embed'])
    species3d = jnp.zeros((N_PAD,), _I32).at[:N_NODES].set(
        species.astype(_I32)).reshape(NT, 1, T)
    pos16 = jnp.zeros((N_NODES, 16), _F32).at[:, :3].set(positions)

    ps = _sc_gather(pos16, send_idx)
    pr = _sc_gather(pos16, recv_idx)
    yT, bT = _geom(ps.T, pr.T)
    yy = yT.T
    bas = bT.T

    h0, sk0 = _emb(species3d, embp, lp0['lin'], lp0['skip'])

    def w01(lp):
        rw2c = lp['rw2'].reshape(64, MUL, 4)[:, :, 0]
        return (lp['rw1'], lp['rb1'].reshape(1, 64), rw2c, lp['self'][0])

    hs0 = _sc_gather(h0, send_idx)
    h1, sk1 = _c01(cs1, cc1, hs0, bas, recv_m, sk0, *w01(lp0),
                   lp1['lin'], lp1['skip'])
    hs1 = _sc_gather(h1, send_idx)
    h2, sk2 = _c01(cs1, cc1, hs1, bas, recv_m, sk1, *w01(lp1),
                   lp2['lin'], lp2['skip'])
    hs2 = _sc_gather(h2, send_idx)

    rw2p = lp2['rw2'].reshape(64, MUL, 4).transpose(0, 2, 1).reshape(64, 256)
    selfs = lp2['self'].reshape(4 * MUL, MUL)
    out2 = _c2(cs2, cc2, hs2, bas, yy, recv_m, sk2,
               lp2['rw1'], lp2['rb1'].reshape(1, 64), rw2p, selfs)
    return out2[:N_NODES].reshape(N_NODES, IRREPS_DIM, MUL).transpose(0, 2, 1)
